# Initial kernel scaffold; baseline (speedup 1.0000x reference)
import jax
import jax.numpy as jnp
from jax.experimental import pallas as pl


def kernel(x, edge_index, W1, b1, W2, b2):
    raise NotImplementedError("write your pallas kernel here")


# speedup vs baseline: 12.8694x; 12.8694x over previous
import functools

import jax
import jax.numpy as jnp
from jax import lax
from jax.experimental import pallas as pl
from jax.experimental.pallas import tpu as pltpu
from jax.experimental.pallas import tpu_sc as plsc

N = 10000
E = 320000
D = 128
NC = 2
NS = 16
EPW = E // (NC * NS)
CH = 80
NCHUNK = EPW // CH
RPT = 632
RPT_LAST = N - (NS - 1) * RPT
N2 = 10240
DPT = N2 // NS

_mesh = plsc.VectorSubcoreMesh(
    core_axis_name="c", subcore_axis_name="s", num_cores=NC, num_subcores=NS
)


@functools.partial(
    pl.kernel,
    out_type=jax.ShapeDtypeStruct((NC, N2), jnp.float32),
    mesh=_mesh,
    scratch_types=[
        pltpu.VMEM((DPT,), jnp.float32),
        pltpu.VMEM((CH,), jnp.float32),
        pltpu.VMEM((2, CH), jnp.int32),
        pltpu.VMEM_SHARED((N2,), jnp.float32),
    ],
)
def _deg_kernel(dst_hbm, out_hbm, half_v, ones_v, idx_v, deg_sh):
    c = lax.axis_index("c")
    s = lax.axis_index("s")
    w = c * NS + s

    def fill_half(i, carry):
        half_v[pl.ds(i * 16, 16)] = jnp.full((16,), 0.5, jnp.float32)
        return carry

    lax.fori_loop(0, DPT // 16, fill_half, 0)

    def fill_one(i, carry):
        ones_v[pl.ds(i * 16, 16)] = jnp.full((16,), 1.0, jnp.float32)
        return carry

    lax.fori_loop(0, CH // 16, fill_one, 0)

    pltpu.sync_copy(half_v, deg_sh.at[pl.ds(s * DPT, DPT)])
    plsc.subcore_barrier()

    def body(j, carry):
        off = w * EPW + j * CH
        pltpu.sync_copy(dst_hbm.at[pl.ds(off, CH)], idx_v.at[0])
        pltpu.sync_copy(ones_v, deg_sh.at[idx_v.at[0]], add=True)
        return carry

    lax.fori_loop(0, NCHUNK, body, 0)

    plsc.subcore_barrier()
    pltpu.sync_copy(deg_sh.at[pl.ds(s * DPT, DPT)], out_hbm.at[c, pl.ds(s * DPT, DPT)])


@functools.partial(
    pl.kernel,
    out_type=jax.ShapeDtypeStruct((NC, N, D), jnp.float32),
    mesh=_mesh,
    scratch_types=[
        pltpu.VMEM((2, CH), jnp.int32),
        pltpu.VMEM((2, CH), jnp.int32),
        pltpu.VMEM((CH, D), jnp.float32),
        pltpu.VMEM_SHARED((N, D), jnp.float32),
        pltpu.SemaphoreType.DMA,
    ],
)
def _edge_kernel(h_hbm, src_hbm, dst_hbm, out_hbm, sidx_v, didx_v, rows_v, acc_sh, gsem):
    c = lax.axis_index("c")
    s = lax.axis_index("s")
    w = c * NS + s

    @pl.when(s < NS - 1)
    def _():
        pltpu.sync_copy(h_hbm.at[pl.ds(s * RPT, RPT)], acc_sh.at[pl.ds(s * RPT, RPT)])

    @pl.when(s == NS - 1)
    def _():
        pltpu.sync_copy(
            h_hbm.at[pl.ds((NS - 1) * RPT, RPT_LAST)],
            acc_sh.at[pl.ds((NS - 1) * RPT, RPT_LAST)],
        )

    plsc.subcore_barrier()

    def body(j, carry):
        off = w * EPW + j * CH
        pltpu.sync_copy(src_hbm.at[pl.ds(off, CH)], sidx_v.at[0])
        pltpu.sync_copy(dst_hbm.at[pl.ds(off, CH)], didx_v.at[0])
        pltpu.async_copy(h_hbm.at[sidx_v.at[0]], rows_v, gsem).wait()
        pltpu.sync_copy(rows_v, acc_sh.at[didx_v.at[0]], add=True)
        return carry

    lax.fori_loop(0, NCHUNK, body, 0)

    plsc.subcore_barrier()

    @pl.when(s < NS - 1)
    def _():
        pltpu.sync_copy(
            acc_sh.at[pl.ds(s * RPT, RPT)], out_hbm.at[c, pl.ds(s * RPT, RPT)]
        )

    @pl.when(s == NS - 1)
    def _():
        pltpu.sync_copy(
            acc_sh.at[pl.ds((NS - 1) * RPT, RPT_LAST)],
            out_hbm.at[c, pl.ds((NS - 1) * RPT, RPT_LAST)],
        )


_RB = 1000


def _mm1_body(x_ref, w_ref, da_ref, db_ref, h_ref, dinv_ref):
    dinv = lax.rsqrt(da_ref[...] + db_ref[...])
    h = jnp.dot(x_ref[...], w_ref[...], preferred_element_type=jnp.float32)
    h_ref[...] = h * dinv
    dinv_ref[...] = dinv


def _mid_body(aa_ref, ab_ref, hp_ref, dinv_ref, b_ref, w_ref, out_ref):
    z = dinv_ref[...] * (aa_ref[...] + ab_ref[...] - hp_ref[...]) + b_ref[...]
    z = jnp.maximum(z, 0.0)
    out_ref[...] = (
        jnp.dot(z, w_ref[...], preferred_element_type=jnp.float32) * dinv_ref[...]
    )


def _fin_body(aa_ref, ab_ref, hp_ref, dinv_ref, b_ref, out_ref):
    out_ref[...] = (
        dinv_ref[...] * (aa_ref[...] + ab_ref[...] - hp_ref[...]) + b_ref[...]
    )


def _row_spec(width):
    return pl.BlockSpec((_RB, width), lambda i: (i, 0))


def _const_spec(shape):
    return pl.BlockSpec(shape, lambda i: (0, 0))


_mm1 = pl.pallas_call(
    _mm1_body,
    grid=(N // _RB,),
    in_specs=[_row_spec(D), _const_spec((D, D)), _row_spec(1), _row_spec(1)],
    out_specs=[_row_spec(D), _row_spec(1)],
    out_shape=[
        jax.ShapeDtypeStruct((N, D), jnp.float32),
        jax.ShapeDtypeStruct((N, 1), jnp.float32),
    ],
)

_mid = pl.pallas_call(
    _mid_body,
    grid=(N // _RB,),
    in_specs=[
        _row_spec(D), _row_spec(D), _row_spec(D), _row_spec(1),
        _const_spec((1, D)), _const_spec((D, D)),
    ],
    out_specs=_row_spec(D),
    out_shape=jax.ShapeDtypeStruct((N, D), jnp.float32),
)

_fin = pl.pallas_call(
    _fin_body,
    grid=(N // _RB,),
    in_specs=[
        _row_spec(D), _row_spec(D), _row_spec(D), _row_spec(1), _const_spec((1, D)),
    ],
    out_specs=_row_spec(D),
    out_shape=jax.ShapeDtypeStruct((N, D), jnp.float32),
)


def kernel(x, edge_index, W1, b1, W2, b2):
    src = edge_index[0].astype(jnp.int32)
    dst = edge_index[1].astype(jnp.int32)

    deg2 = _deg_kernel(dst)
    da = deg2[0, :N].reshape(N, 1)
    db = deg2[1, :N].reshape(N, 1)

    h1p, dinv = _mm1(x, W1, da, db)
    acc1 = _edge_kernel(h1p, src, dst)
    h2p = _mid(acc1[0], acc1[1], h1p, dinv, b1.reshape(1, D), W2)
    acc2 = _edge_kernel(h2p, src, dst)
    return _fin(acc2[0], acc2[1], h2p, dinv, b2.reshape(1, D))


# speedup vs baseline: 26.0015x; 2.0204x over previous
import functools

import jax
import jax.numpy as jnp
from jax import lax
from jax.experimental import pallas as pl
from jax.experimental.pallas import tpu as pltpu
from jax.experimental.pallas import tpu_sc as plsc

N = 10000
E = 320000
D = 128
NC = 2
NS = 16
NW = NC * NS
CH = 125
NCHT = E // CH
NCHUNK = NCHT // NW
UNROLL = 4
NITER = NCHUNK // UNROLL
RPT = 632
RPT_LAST = N - (NS - 1) * RPT
N2 = 10240
DPT = N2 // NS

_mesh = plsc.VectorSubcoreMesh(
    core_axis_name="c", subcore_axis_name="s", num_cores=NC, num_subcores=NS
)


@functools.partial(
    pl.kernel,
    out_type=jax.ShapeDtypeStruct((NC, N2), jnp.float32),
    mesh=_mesh,
    scratch_types=[
        pltpu.VMEM((DPT,), jnp.float32),
        pltpu.VMEM((128,), jnp.float32),
        pltpu.VMEM((2, CH), jnp.int32),
        pltpu.VMEM_SHARED((N2,), jnp.float32),
        pltpu.SemaphoreType.DMA,
    ],
)
def _deg_kernel(idx_hbm, out_hbm, half_v, ones_v, idx_v, deg_sh, isem):
    c = lax.axis_index("c")
    s = lax.axis_index("s")
    w = c * NS + s
    base = w * NCHUNK

    def fill_half(i, carry):
        half_v[pl.ds(i * 16, 16)] = jnp.full((16,), 0.5, jnp.float32)
        return carry

    lax.fori_loop(0, DPT // 16, fill_half, 0)

    def fill_one(i, carry):
        ones_v[pl.ds(i * 16, 16)] = jnp.full((16,), 1.0, jnp.float32)
        return carry

    lax.fori_loop(0, 128 // 16, fill_one, 0)

    pltpu.sync_copy(half_v, deg_sh.at[pl.ds(s * DPT, DPT)])
    pltpu.async_copy(idx_hbm.at[base, 1], idx_v.at[0], isem).wait()
    plsc.subcore_barrier()

    def body(t, carry):
        for u in range(2):
            k = t * 2 + u

            def chunk(do_pf):
                if do_pf:
                    d = pltpu.async_copy(
                        idx_hbm.at[base + k + 1, 1], idx_v.at[1 - u], isem
                    )
                pltpu.sync_copy(
                    ones_v.at[pl.ds(0, CH)], deg_sh.at[idx_v.at[u]], add=True
                )
                if do_pf:
                    d.wait()

            if u == 0:
                chunk(True)
            else:
                pl.when(t < NCHUNK // 2 - 1)(lambda: chunk(True))
                pl.when(t == NCHUNK // 2 - 1)(lambda: chunk(False))
        return carry

    lax.fori_loop(0, NCHUNK // 2, body, 0)

    plsc.subcore_barrier()
    pltpu.sync_copy(deg_sh.at[pl.ds(s * DPT, DPT)], out_hbm.at[c, pl.ds(s * DPT, DPT)])


@functools.partial(
    pl.kernel,
    out_type=jax.ShapeDtypeStruct((NC, N, D), jnp.float32),
    mesh=_mesh,
    scratch_types=[
        pltpu.VMEM((UNROLL, 2, CH), jnp.int32),
        pltpu.VMEM((2, CH, D), jnp.float32),
        pltpu.VMEM_SHARED((N, D), jnp.float32),
        pltpu.SemaphoreType.DMA,
        pltpu.SemaphoreType.DMA,
        pltpu.SemaphoreType.DMA,
    ],
)
def _edge_kernel(h_hbm, idx_hbm, out_hbm, idx_v, rows_v, acc_sh, isem, gsem, asem):
    c = lax.axis_index("c")
    s = lax.axis_index("s")
    w = c * NS + s
    base = w * NCHUNK

    def prologue(row0, nrows):
        ainit = pltpu.async_copy(
            h_hbm.at[pl.ds(row0, nrows)], acc_sh.at[pl.ds(row0, nrows)], asem
        )
        ip = pltpu.async_copy(
            idx_hbm.at[pl.ds(base, 2)], idx_v.at[pl.ds(0, 2)], isem
        )
        ip.wait()
        pltpu.async_copy(h_hbm.at[idx_v.at[0, 0]], rows_v.at[0], gsem).wait()
        ainit.wait()

    pl.when(s < NS - 1)(lambda: prologue(s * RPT, RPT))
    pl.when(s == NS - 1)(lambda: prologue((NS - 1) * RPT, RPT_LAST))

    plsc.subcore_barrier()

    def body(t, carry):
        for u in range(UNROLL):
            k = t * UNROLL + u
            b = u % 2
            q, q1, q2 = u, (u + 1) % UNROLL, (u + 2) % UNROLL

            def chunk(do_pf, do_g):
                if do_pf:
                    dpf = pltpu.async_copy(
                        idx_hbm.at[base + k + 2], idx_v.at[q2], isem
                    )
                if do_g:
                    dg = pltpu.async_copy(
                        h_hbm.at[idx_v.at[q1, 0]], rows_v.at[1 - b], gsem
                    )
                pltpu.sync_copy(rows_v.at[b], acc_sh.at[idx_v.at[q, 1]], add=True)
                if do_g:
                    dg.wait()
                if do_pf:
                    dpf.wait()

            if u < 2:
                chunk(True, True)
            elif u == 2:
                pl.when(t < NITER - 1)(lambda: chunk(True, True))
                pl.when(t == NITER - 1)(lambda: chunk(False, True))
            else:
                pl.when(t < NITER - 1)(lambda: chunk(True, True))
                pl.when(t == NITER - 1)(lambda: chunk(False, False))
        return carry

    lax.fori_loop(0, NITER, body, 0)

    plsc.subcore_barrier()

    @pl.when(s < NS - 1)
    def _():
        pltpu.sync_copy(
            acc_sh.at[pl.ds(s * RPT, RPT)], out_hbm.at[c, pl.ds(s * RPT, RPT)]
        )

    @pl.when(s == NS - 1)
    def _():
        pltpu.sync_copy(
            acc_sh.at[pl.ds((NS - 1) * RPT, RPT_LAST)],
            out_hbm.at[c, pl.ds((NS - 1) * RPT, RPT_LAST)],
        )


_RB = 1000


def _mm1_body(x_ref, w_ref, da_ref, db_ref, h_ref, dinv_ref):
    dinv = lax.rsqrt(da_ref[...] + db_ref[...])
    h = jnp.dot(x_ref[...], w_ref[...], preferred_element_type=jnp.float32)
    h_ref[...] = h * dinv
    dinv_ref[...] = dinv


def _mid_body(aa_ref, ab_ref, hp_ref, dinv_ref, b_ref, w_ref, out_ref):
    z = dinv_ref[...] * (aa_ref[...] + ab_ref[...] - hp_ref[...]) + b_ref[...]
    z = jnp.maximum(z, 0.0)
    out_ref[...] = (
        jnp.dot(z, w_ref[...], preferred_element_type=jnp.float32) * dinv_ref[...]
    )


def _fin_body(aa_ref, ab_ref, hp_ref, dinv_ref, b_ref, out_ref):
    out_ref[...] = (
        dinv_ref[...] * (aa_ref[...] + ab_ref[...] - hp_ref[...]) + b_ref[...]
    )


def _row_spec(width):
    return pl.BlockSpec((_RB, width), lambda i: (i, 0))


def _const_spec(shape):
    return pl.BlockSpec(shape, lambda i: (0, 0))


_mm1 = pl.pallas_call(
    _mm1_body,
    grid=(N // _RB,),
    in_specs=[_row_spec(D), _const_spec((D, D)), _row_spec(1), _row_spec(1)],
    out_specs=[_row_spec(D), _row_spec(1)],
    out_shape=[
        jax.ShapeDtypeStruct((N, D), jnp.float32),
        jax.ShapeDtypeStruct((N, 1), jnp.float32),
    ],
)

_mid = pl.pallas_call(
    _mid_body,
    grid=(N // _RB,),
    in_specs=[
        _row_spec(D), _row_spec(D), _row_spec(D), _row_spec(1),
        _const_spec((1, D)), _const_spec((D, D)),
    ],
    out_specs=_row_spec(D),
    out_shape=jax.ShapeDtypeStruct((N, D), jnp.float32),
)

_fin = pl.pallas_call(
    _fin_body,
    grid=(N // _RB,),
    in_specs=[
        _row_spec(D), _row_spec(D), _row_spec(D), _row_spec(1), _const_spec((1, D)),
    ],
    out_specs=_row_spec(D),
    out_shape=jax.ShapeDtypeStruct((N, D), jnp.float32),
)


def kernel(x, edge_index, W1, b1, W2, b2):
    idx3 = edge_index.astype(jnp.int32).reshape(2, NCHT, CH).transpose(1, 0, 2)

    deg2 = _deg_kernel(idx3)
    da = deg2[0, :N].reshape(N, 1)
    db = deg2[1, :N].reshape(N, 1)

    h1p, dinv = _mm1(x, W1, da, db)
    acc1 = _edge_kernel(h1p, idx3)
    h2p = _mid(acc1[0], acc1[1], h1p, dinv, b1.reshape(1, D), W2)
    acc2 = _edge_kernel(h2p, idx3)
    return _fin(acc2[0], acc2[1], h2p, dinv, b2.reshape(1, D))


# speedup vs baseline: 31.8469x; 1.2248x over previous
import functools

import jax
import jax.numpy as jnp
from jax import lax
from jax.experimental import pallas as pl
from jax.experimental.pallas import tpu as pltpu
from jax.experimental.pallas import tpu_sc as plsc

N = 10000
E = 320000
D = 128
NC = 2
NS = 16
NW = NC * NS
CH = 125
NCHT = E // CH
NCHUNK = NCHT // NW
UNROLL = 4
NITER = NCHUNK // UNROLL
RPT = 632
RPT_LAST = N - (NS - 1) * RPT
N2 = 10240
DPT = N2 // NS

_mesh = plsc.VectorSubcoreMesh(
    core_axis_name="c", subcore_axis_name="s", num_cores=NC, num_subcores=NS
)


@functools.partial(
    pl.kernel,
    out_type=jax.ShapeDtypeStruct((NC, N2), jnp.float32),
    mesh=_mesh,
    scratch_types=[
        pltpu.VMEM((DPT,), jnp.float32),
        pltpu.VMEM((128,), jnp.float32),
        pltpu.VMEM((2, CH), jnp.int32),
        pltpu.VMEM_SHARED((N2,), jnp.float32),
        pltpu.SemaphoreType.DMA,
    ],
)
def _deg_kernel(idx_hbm, out_hbm, half_v, ones_v, idx_v, deg_sh, isem):
    c = lax.axis_index("c")
    s = lax.axis_index("s")
    w = c * NS + s
    base = w * NCHUNK

    def fill_half(i, carry):
        half_v[pl.ds(i * 16, 16)] = jnp.full((16,), 0.5, jnp.float32)
        return carry

    lax.fori_loop(0, DPT // 16, fill_half, 0)

    def fill_one(i, carry):
        ones_v[pl.ds(i * 16, 16)] = jnp.full((16,), 1.0, jnp.float32)
        return carry

    lax.fori_loop(0, 128 // 16, fill_one, 0)

    pltpu.sync_copy(half_v, deg_sh.at[pl.ds(s * DPT, DPT)])
    pltpu.async_copy(idx_hbm.at[base, 1], idx_v.at[0], isem).wait()
    plsc.subcore_barrier()

    def body(t, carry):
        for u in range(2):
            k = t * 2 + u

            def chunk(do_pf):
                if do_pf:
                    d = pltpu.async_copy(
                        idx_hbm.at[base + k + 1, 1], idx_v.at[1 - u], isem
                    )
                pltpu.sync_copy(
                    ones_v.at[pl.ds(0, CH)], deg_sh.at[idx_v.at[u]], add=True
                )
                if do_pf:
                    d.wait()

            if u == 0:
                chunk(True)
            else:
                pl.when(t < NCHUNK // 2 - 1)(lambda: chunk(True))
                pl.when(t == NCHUNK // 2 - 1)(lambda: chunk(False))
        return carry

    lax.fori_loop(0, NCHUNK // 2, body, 0)

    plsc.subcore_barrier()
    pltpu.sync_copy(deg_sh.at[pl.ds(s * DPT, DPT)], out_hbm.at[c, pl.ds(s * DPT, DPT)])


@functools.partial(
    pl.kernel,
    out_type=jax.ShapeDtypeStruct((NC, N, D), jnp.float32),
    mesh=_mesh,
    scratch_types=[
        pltpu.VMEM((UNROLL, 2, CH), jnp.int32),
        pltpu.VMEM((3, CH, D), jnp.float32),
        pltpu.VMEM_SHARED((N, D), jnp.float32),
        pltpu.SemaphoreType.DMA,
        pltpu.SemaphoreType.DMA,
        pltpu.SemaphoreType.DMA,
        pltpu.SemaphoreType.DMA,
    ],
)
def _edge_kernel(
    h_hbm, idx_hbm, out_hbm, idx_v, rows_v, acc_sh, isem, gsemA, gsemB, asem
):
    c = lax.axis_index("c")
    s = lax.axis_index("s")
    w = c * NS + s
    base = w * NCHUNK

    def prologue(row0, nrows):
        ainit = pltpu.async_copy(
            h_hbm.at[pl.ds(row0, nrows)], acc_sh.at[pl.ds(row0, nrows)], asem
        )
        ip = pltpu.async_copy(
            idx_hbm.at[pl.ds(base, 3)], idx_v.at[pl.ds(0, 3)], isem
        )
        ip.wait()
        g0 = pltpu.async_copy(h_hbm.at[idx_v.at[0, 0]], rows_v.at[0], gsemA)
        pltpu.async_copy(h_hbm.at[idx_v.at[1, 0]], rows_v.at[1], gsemB)
        g0.wait()
        ainit.wait()


    pl.when(s < NS - 1)(lambda: prologue(s * RPT, RPT))
    pl.when(s == NS - 1)(lambda: prologue((NS - 1) * RPT, RPT_LAST))

    plsc.subcore_barrier()

    def body(t, carry):
        for u in range(UNROLL):
            k = t * UNROLL + u
            q, q1, q2, q3 = u, (u + 1) % 4, (u + 2) % 4, (u + 3) % 4
            r = lax.rem(k, 3)
            r1 = lax.rem(k + 1, 3)
            r2 = lax.rem(k + 2, 3)
            gs_issue = (gsemA, gsemB)[u % 2]
            gs_wait = (gsemA, gsemB)[(u + 1) % 2]

            def chunk(do_pf, do_g2, do_w1):
                if do_pf:
                    dpf = pltpu.async_copy(
                        idx_hbm.at[base + k + 3], idx_v.at[q3], isem
                    )
                if do_g2:
                    pltpu.async_copy(
                        h_hbm.at[idx_v.at[q2, 0]], rows_v.at[r2], gs_issue
                    )
                pltpu.sync_copy(rows_v.at[r], acc_sh.at[idx_v.at[q, 1]], add=True)
                if do_w1:
                    pltpu.make_async_copy(
                        h_hbm.at[idx_v.at[q1, 0]], rows_v.at[r1], gs_wait
                    ).wait()
                if do_pf:
                    dpf.wait()

            if u == 0:
                chunk(True, True, True)
            else:
                tails = {1: (False, True, True), 2: (False, False, True),
                         3: (False, False, False)}[u]
                pl.when(t < NITER - 1)(lambda: chunk(True, True, True))
                pl.when(t == NITER - 1)(lambda: chunk(*tails))
        return carry

    lax.fori_loop(0, NITER, body, 0)

    plsc.subcore_barrier()

    @pl.when(s < NS - 1)
    def _():
        pltpu.sync_copy(
            acc_sh.at[pl.ds(s * RPT, RPT)], out_hbm.at[c, pl.ds(s * RPT, RPT)]
        )

    @pl.when(s == NS - 1)
    def _():
        pltpu.sync_copy(
            acc_sh.at[pl.ds((NS - 1) * RPT, RPT_LAST)],
            out_hbm.at[c, pl.ds((NS - 1) * RPT, RPT_LAST)],
        )


_RB = 1000


def _mm1_body(x_ref, w_ref, da_ref, db_ref, h_ref, dinv_ref):
    dinv = lax.rsqrt(da_ref[...] + db_ref[...])
    h = jnp.dot(x_ref[...], w_ref[...], preferred_element_type=jnp.float32)
    h_ref[...] = h * dinv
    dinv_ref[...] = dinv


def _mid_body(aa_ref, ab_ref, hp_ref, dinv_ref, b_ref, w_ref, out_ref):
    z = dinv_ref[...] * (aa_ref[...] + ab_ref[...] - hp_ref[...]) + b_ref[...]
    z = jnp.maximum(z, 0.0)
    out_ref[...] = (
        jnp.dot(z, w_ref[...], preferred_element_type=jnp.float32) * dinv_ref[...]
    )


def _fin_body(aa_ref, ab_ref, hp_ref, dinv_ref, b_ref, out_ref):
    out_ref[...] = (
        dinv_ref[...] * (aa_ref[...] + ab_ref[...] - hp_ref[...]) + b_ref[...]
    )


def _row_spec(width):
    return pl.BlockSpec((_RB, width), lambda i: (i, 0))


def _const_spec(shape):
    return pl.BlockSpec(shape, lambda i: (0, 0))


_mm1 = pl.pallas_call(
    _mm1_body,
    grid=(N // _RB,),
    in_specs=[_row_spec(D), _const_spec((D, D)), _row_spec(1), _row_spec(1)],
    out_specs=[_row_spec(D), _row_spec(1)],
    out_shape=[
        jax.ShapeDtypeStruct((N, D), jnp.float32),
        jax.ShapeDtypeStruct((N, 1), jnp.float32),
    ],
)

_mid = pl.pallas_call(
    _mid_body,
    grid=(N // _RB,),
    in_specs=[
        _row_spec(D), _row_spec(D), _row_spec(D), _row_spec(1),
        _const_spec((1, D)), _const_spec((D, D)),
    ],
    out_specs=_row_spec(D),
    out_shape=jax.ShapeDtypeStruct((N, D), jnp.float32),
)

_fin = pl.pallas_call(
    _fin_body,
    grid=(N // _RB,),
    in_specs=[
        _row_spec(D), _row_spec(D), _row_spec(D), _row_spec(1), _const_spec((1, D)),
    ],
    out_specs=_row_spec(D),
    out_shape=jax.ShapeDtypeStruct((N, D), jnp.float32),
)


def kernel(x, edge_index, W1, b1, W2, b2):
    idx3 = edge_index.astype(jnp.int32).reshape(2, NCHT, CH).transpose(1, 0, 2)

    deg2 = _deg_kernel(idx3)
    da = deg2[0, :N].reshape(N, 1)
    db = deg2[1, :N].reshape(N, 1)

    h1p, dinv = _mm1(x, W1, da, db)
    acc1 = _edge_kernel(h1p, idx3)
    h2p = _mid(acc1[0], acc1[1], h1p, dinv, b1.reshape(1, D), W2)
    acc2 = _edge_kernel(h2p, idx3)
    return _fin(acc2[0], acc2[1], h2p, dinv, b2.reshape(1, D))


# speedup vs baseline: 32.3604x; 1.0161x over previous
import functools

import jax
import jax.numpy as jnp
from jax import lax
from jax.experimental import pallas as pl
from jax.experimental.pallas import tpu as pltpu
from jax.experimental.pallas import tpu_sc as plsc

N = 10000
E = 320000
D = 128
NC = 2
NS = 16
NW = NC * NS
CH = 125
NCHT = E // CH
NCHUNK = NCHT // NW
UNROLL = 4
NITER = NCHUNK // UNROLL
RPT = 632
RPT_LAST = N - (NS - 1) * RPT
N2 = 10240
DPT = N2 // NS

_mesh = plsc.VectorSubcoreMesh(
    core_axis_name="c", subcore_axis_name="s", num_cores=NC, num_subcores=NS
)


CHD = 80
NCHD = E // (NW * CHD)


@functools.partial(
    pl.kernel,
    out_type=jax.ShapeDtypeStruct((NC, N2), jnp.float32),
    mesh=_mesh,
    scratch_types=[
        pltpu.VMEM((DPT,), jnp.float32),
        pltpu.VMEM((CHD,), jnp.float32),
        pltpu.VMEM((2, CHD), jnp.int32),
        pltpu.VMEM_SHARED((N2,), jnp.float32),
        pltpu.SemaphoreType.DMA,
    ],
)
def _deg_kernel(dst_hbm, out_hbm, half_v, ones_v, idx_v, deg_sh, isem):
    c = lax.axis_index("c")
    s = lax.axis_index("s")
    w = c * NS + s
    base = w * NCHD * CHD

    def fill_half(i, carry):
        half_v[pl.ds(i * 16, 16)] = jnp.full((16,), 0.5, jnp.float32)
        return carry

    lax.fori_loop(0, DPT // 16, fill_half, 0)

    def fill_one(i, carry):
        ones_v[pl.ds(i * 16, 16)] = jnp.full((16,), 1.0, jnp.float32)
        return carry

    lax.fori_loop(0, CHD // 16, fill_one, 0)

    pltpu.sync_copy(half_v, deg_sh.at[pl.ds(s * DPT, DPT)])
    pltpu.async_copy(dst_hbm.at[pl.ds(base, CHD)], idx_v.at[0], isem).wait()
    plsc.subcore_barrier()

    def body(k, carry):
        r = lax.rem(k, 2)
        r1 = lax.rem(k + 1, 2)

        @pl.when(k < NCHD - 1)
        def _():
            d = pltpu.async_copy(
                dst_hbm.at[pl.ds(base + (k + 1) * CHD, CHD)], idx_v.at[r1], isem
            )
            pltpu.sync_copy(ones_v, deg_sh.at[idx_v.at[r]], add=True)
            d.wait()

        @pl.when(k == NCHD - 1)
        def _():
            pltpu.sync_copy(ones_v, deg_sh.at[idx_v.at[r]], add=True)

        return carry

    lax.fori_loop(0, NCHD, body, 0)

    plsc.subcore_barrier()
    pltpu.sync_copy(deg_sh.at[pl.ds(s * DPT, DPT)], out_hbm.at[c, pl.ds(s * DPT, DPT)])


@functools.partial(
    pl.kernel,
    out_type=jax.ShapeDtypeStruct((NC, N, D), jnp.float32),
    mesh=_mesh,
    scratch_types=[
        pltpu.VMEM((UNROLL, 2, CH), jnp.int32),
        pltpu.VMEM((3, CH, D), jnp.float32),
        pltpu.VMEM_SHARED((N, D), jnp.float32),
        pltpu.SemaphoreType.DMA,
        pltpu.SemaphoreType.DMA,
        pltpu.SemaphoreType.DMA,
        pltpu.SemaphoreType.DMA,
    ],
)
def _edge_kernel(
    h_hbm, idx_hbm, out_hbm, idx_v, rows_v, acc_sh, isem, gsemA, gsemB, asem
):
    c = lax.axis_index("c")
    s = lax.axis_index("s")
    w = c * NS + s
    base = w * NCHUNK

    def prologue(row0, nrows):
        ainit = pltpu.async_copy(
            h_hbm.at[pl.ds(row0, nrows)], acc_sh.at[pl.ds(row0, nrows)], asem
        )
        ip = pltpu.async_copy(
            idx_hbm.at[pl.ds(base, 3)], idx_v.at[pl.ds(0, 3)], isem
        )
        ip.wait()
        g0 = pltpu.async_copy(h_hbm.at[idx_v.at[0, 0]], rows_v.at[0], gsemA)
        pltpu.async_copy(h_hbm.at[idx_v.at[1, 0]], rows_v.at[1], gsemB)
        g0.wait()
        ainit.wait()


    pl.when(s < NS - 1)(lambda: prologue(s * RPT, RPT))
    pl.when(s == NS - 1)(lambda: prologue((NS - 1) * RPT, RPT_LAST))

    plsc.subcore_barrier()

    def body(t, carry):
        for u in range(UNROLL):
            k = t * UNROLL + u
            q, q1, q2, q3 = u, (u + 1) % 4, (u + 2) % 4, (u + 3) % 4
            r = lax.rem(k, 3)
            r1 = lax.rem(k + 1, 3)
            r2 = lax.rem(k + 2, 3)
            gs_issue = (gsemA, gsemB)[u % 2]
            gs_wait = (gsemA, gsemB)[(u + 1) % 2]

            def chunk(do_pf, do_g2, do_w1):
                if do_pf:
                    dpf = pltpu.async_copy(
                        idx_hbm.at[base + k + 3], idx_v.at[q3], isem
                    )
                if do_g2:
                    pltpu.async_copy(
                        h_hbm.at[idx_v.at[q2, 0]], rows_v.at[r2], gs_issue
                    )
                pltpu.sync_copy(rows_v.at[r], acc_sh.at[idx_v.at[q, 1]], add=True)
                if do_w1:
                    pltpu.make_async_copy(
                        h_hbm.at[idx_v.at[q1, 0]], rows_v.at[r1], gs_wait
                    ).wait()
                if do_pf:
                    dpf.wait()

            if u == 0:
                chunk(True, True, True)
            else:
                tails = {1: (False, True, True), 2: (False, False, True),
                         3: (False, False, False)}[u]
                pl.when(t < NITER - 1)(lambda: chunk(True, True, True))
                pl.when(t == NITER - 1)(lambda: chunk(*tails))
        return carry

    lax.fori_loop(0, NITER, body, 0)

    plsc.subcore_barrier()

    @pl.when(s < NS - 1)
    def _():
        pltpu.sync_copy(
            acc_sh.at[pl.ds(s * RPT, RPT)], out_hbm.at[c, pl.ds(s * RPT, RPT)]
        )

    @pl.when(s == NS - 1)
    def _():
        pltpu.sync_copy(
            acc_sh.at[pl.ds((NS - 1) * RPT, RPT_LAST)],
            out_hbm.at[c, pl.ds((NS - 1) * RPT, RPT_LAST)],
        )


_RB = 1000


def _mm1_body(x_ref, w_ref, dg_ref, h_ref, dinv_ref):
    dinv = lax.rsqrt(dg_ref[:, 0:1] + dg_ref[:, 1:2])
    h = jnp.dot(x_ref[...], w_ref[...], preferred_element_type=jnp.float32)
    h_ref[...] = h * dinv
    dinv_ref[...] = dinv


def _mid_body(acc_ref, hp_ref, dinv_ref, b_ref, w_ref, out_ref):
    z = (
        dinv_ref[...] * (acc_ref[0] + acc_ref[1] - hp_ref[...]) + b_ref[...]
    )
    z = jnp.maximum(z, 0.0)
    out_ref[...] = (
        jnp.dot(z, w_ref[...], preferred_element_type=jnp.float32) * dinv_ref[...]
    )


def _fin_body(acc_ref, hp_ref, dinv_ref, b_ref, out_ref):
    out_ref[...] = (
        dinv_ref[...] * (acc_ref[0] + acc_ref[1] - hp_ref[...]) + b_ref[...]
    )


def _row_spec(width):
    return pl.BlockSpec((_RB, width), lambda i: (i, 0))


def _acc_spec():
    return pl.BlockSpec((2, _RB, D), lambda i: (0, i, 0))


def _const_spec(shape):
    return pl.BlockSpec(shape, lambda i: (0, 0))


_mm1 = pl.pallas_call(
    _mm1_body,
    grid=(N // _RB,),
    in_specs=[_row_spec(D), _const_spec((D, D)), _row_spec(2)],
    out_specs=[_row_spec(D), _row_spec(1)],
    out_shape=[
        jax.ShapeDtypeStruct((N, D), jnp.float32),
        jax.ShapeDtypeStruct((N, 1), jnp.float32),
    ],
)

_mid = pl.pallas_call(
    _mid_body,
    grid=(N // _RB,),
    in_specs=[
        _acc_spec(), _row_spec(D), _row_spec(1),
        _const_spec((1, D)), _const_spec((D, D)),
    ],
    out_specs=_row_spec(D),
    out_shape=jax.ShapeDtypeStruct((N, D), jnp.float32),
)

_fin = pl.pallas_call(
    _fin_body,
    grid=(N // _RB,),
    in_specs=[
        _acc_spec(), _row_spec(D), _row_spec(1), _const_spec((1, D)),
    ],
    out_specs=_row_spec(D),
    out_shape=jax.ShapeDtypeStruct((N, D), jnp.float32),
)


def kernel(x, edge_index, W1, b1, W2, b2):
    ei = edge_index.astype(jnp.int32)
    idx3 = ei.reshape(2, NCHT, CH).transpose(1, 0, 2)

    deg2 = _deg_kernel(ei[1])
    degT = deg2.T[:N]

    h1p, dinv = _mm1(x, W1, degT)
    acc1 = _edge_kernel(h1p, idx3)
    h2p = _mid(acc1, h1p, dinv, b1.reshape(1, D), W2)
    acc2 = _edge_kernel(h2p, idx3)
    return _fin(acc2, h2p, dinv, b2.reshape(1, D))


# speedup vs baseline: 35.5481x; 1.0985x over previous
import functools

import jax
import jax.numpy as jnp
from jax import lax
from jax.experimental import pallas as pl
from jax.experimental.pallas import tpu as pltpu
from jax.experimental.pallas import tpu_sc as plsc

N = 10000
E = 320000
D = 128
NC = 2
NS = 16
NW = NC * NS
CH = 125
NCHT = E // CH
NCHUNK = NCHT // NW
UNROLL = 4
NITER = NCHUNK // UNROLL
RPT = 632
RPT_LAST = N - (NS - 1) * RPT
N2 = 10240
DPT = N2 // NS

_mesh = plsc.VectorSubcoreMesh(
    core_axis_name="c", subcore_axis_name="s", num_cores=NC, num_subcores=NS
)


CHD = 128
NCHD_TOT = E // CHD
NCHD = (NCHD_TOT + NW - 1) // NW


@functools.partial(
    pl.kernel,
    out_type=jax.ShapeDtypeStruct((NC, N2), jnp.float32),
    mesh=_mesh,
    scratch_types=[
        pltpu.VMEM((DPT,), jnp.float32),
        pltpu.VMEM((CHD,), jnp.float32),
        pltpu.VMEM((2, CHD), jnp.int32),
        pltpu.VMEM_SHARED((N2,), jnp.float32),
        pltpu.SemaphoreType.DMA,
    ],
)
def _deg_kernel(ei_hbm, out_hbm, half_v, ones_v, idx_v, deg_sh, isem):
    c = lax.axis_index("c")
    s = lax.axis_index("s")
    w = c * NS + s

    def fill_half(i, carry):
        half_v[pl.ds(i * 16, 16)] = jnp.full((16,), 0.5, jnp.float32)
        return carry

    lax.fori_loop(0, DPT // 16, fill_half, 0)

    def fill_one(i, carry):
        ones_v[pl.ds(i * 16, 16)] = jnp.full((16,), 1.0, jnp.float32)
        return carry

    lax.fori_loop(0, CHD // 16, fill_one, 0)

    pltpu.sync_copy(half_v, deg_sh.at[pl.ds(s * DPT, DPT)])
    pltpu.async_copy(ei_hbm.at[1, pl.ds(w * CHD, CHD)], idx_v.at[0], isem).wait()
    plsc.subcore_barrier()

    def body(k, carry):
        r = lax.rem(k, 2)
        r1 = lax.rem(k + 1, 2)
        nxt = w + (k + 1) * NW

        @pl.when(nxt < NCHD_TOT)
        def _():
            d = pltpu.async_copy(
                ei_hbm.at[1, pl.ds(nxt * CHD, CHD)], idx_v.at[r1], isem
            )
            pltpu.sync_copy(ones_v, deg_sh.at[idx_v.at[r]], add=True)
            d.wait()

        @pl.when(jnp.logical_and(nxt >= NCHD_TOT, w + k * NW < NCHD_TOT))
        def _():
            pltpu.sync_copy(ones_v, deg_sh.at[idx_v.at[r]], add=True)

        return carry

    lax.fori_loop(0, NCHD, body, 0)

    plsc.subcore_barrier()
    pltpu.sync_copy(
        deg_sh.at[pl.ds(s * DPT, DPT)], out_hbm.at[c, pl.ds(s * DPT, DPT)]
    )


@functools.partial(
    pl.kernel,
    out_type=jax.ShapeDtypeStruct((NC, N, D), jnp.float32),
    mesh=_mesh,
    scratch_types=[
        pltpu.VMEM((UNROLL, 2, CH), jnp.int32),
        pltpu.VMEM((3, CH, D), jnp.float32),
        pltpu.VMEM_SHARED((N, D), jnp.float32),
        pltpu.SemaphoreType.DMA,
        pltpu.SemaphoreType.DMA,
        pltpu.SemaphoreType.DMA,
        pltpu.SemaphoreType.DMA,
    ],
)
def _edge_kernel(
    h_hbm, idx_hbm, out_hbm, idx_v, rows_v, acc_sh, isem, gsemA, gsemB, asem
):
    c = lax.axis_index("c")
    s = lax.axis_index("s")
    w = c * NS + s
    base = w * NCHUNK

    def prologue(row0, nrows):
        ainit = pltpu.async_copy(
            h_hbm.at[pl.ds(row0, nrows)], acc_sh.at[pl.ds(row0, nrows)], asem
        )
        ip = pltpu.async_copy(
            idx_hbm.at[pl.ds(base, 3)], idx_v.at[pl.ds(0, 3)], isem
        )
        ip.wait()
        g0 = pltpu.async_copy(h_hbm.at[idx_v.at[0, 0]], rows_v.at[0], gsemA)
        pltpu.async_copy(h_hbm.at[idx_v.at[1, 0]], rows_v.at[1], gsemB)
        g0.wait()
        ainit.wait()


    pl.when(s < NS - 1)(lambda: prologue(s * RPT, RPT))
    pl.when(s == NS - 1)(lambda: prologue((NS - 1) * RPT, RPT_LAST))

    plsc.subcore_barrier()

    def body(t, carry):
        for u in range(UNROLL):
            k = t * UNROLL + u
            q, q1, q2, q3 = u, (u + 1) % 4, (u + 2) % 4, (u + 3) % 4
            r = lax.rem(k, 3)
            r1 = lax.rem(k + 1, 3)
            r2 = lax.rem(k + 2, 3)
            gs_issue = (gsemA, gsemB)[u % 2]
            gs_wait = (gsemA, gsemB)[(u + 1) % 2]

            def chunk(do_pf, do_g2, do_w1):
                if do_pf:
                    dpf = pltpu.async_copy(
                        idx_hbm.at[base + k + 3], idx_v.at[q3], isem
                    )
                if do_g2:
                    pltpu.async_copy(
                        h_hbm.at[idx_v.at[q2, 0]], rows_v.at[r2], gs_issue
                    )
                pltpu.sync_copy(rows_v.at[r], acc_sh.at[idx_v.at[q, 1]], add=True)
                if do_w1:
                    pltpu.make_async_copy(
                        h_hbm.at[idx_v.at[q1, 0]], rows_v.at[r1], gs_wait
                    ).wait()
                if do_pf:
                    dpf.wait()

            if u == 0:
                chunk(True, True, True)
            else:
                tails = {1: (False, True, True), 2: (False, False, True),
                         3: (False, False, False)}[u]
                pl.when(t < NITER - 1)(lambda: chunk(True, True, True))
                pl.when(t == NITER - 1)(lambda: chunk(*tails))
        return carry

    lax.fori_loop(0, NITER, body, 0)

    plsc.subcore_barrier()

    @pl.when(s < NS - 1)
    def _():
        pltpu.sync_copy(
            acc_sh.at[pl.ds(s * RPT, RPT)], out_hbm.at[c, pl.ds(s * RPT, RPT)]
        )

    @pl.when(s == NS - 1)
    def _():
        pltpu.sync_copy(
            acc_sh.at[pl.ds((NS - 1) * RPT, RPT_LAST)],
            out_hbm.at[c, pl.ds((NS - 1) * RPT, RPT_LAST)],
        )


_RB = 1000


def _mm0_body(x_ref, w_ref, u_ref):
    u_ref[...] = jnp.dot(x_ref[...], w_ref[...], preferred_element_type=jnp.float32)


def _scale_body(u_ref, dg_ref, h_ref, dinv_ref):
    dinv = lax.rsqrt(dg_ref[:, 0:1] + dg_ref[:, 1:2])
    h_ref[...] = u_ref[...] * dinv
    dinv_ref[...] = dinv


def _mid_body(acc_ref, hp_ref, dinv_ref, b_ref, w_ref, out_ref):
    z = (
        dinv_ref[...] * (acc_ref[0] + acc_ref[1] - hp_ref[...]) + b_ref[...]
    )
    z = jnp.maximum(z, 0.0)
    out_ref[...] = (
        jnp.dot(z, w_ref[...], preferred_element_type=jnp.float32) * dinv_ref[...]
    )


def _fin_body(acc_ref, hp_ref, dinv_ref, b_ref, out_ref):
    out_ref[...] = (
        dinv_ref[...] * (acc_ref[0] + acc_ref[1] - hp_ref[...]) + b_ref[...]
    )


def _row_spec(width):
    return pl.BlockSpec((_RB, width), lambda i: (i, 0))


def _acc_spec():
    return pl.BlockSpec((2, _RB, D), lambda i: (0, i, 0))


def _const_spec(shape):
    return pl.BlockSpec(shape, lambda i: (0, 0))


_mm0 = pl.pallas_call(
    _mm0_body,
    grid=(N // _RB,),
    in_specs=[_row_spec(D), _const_spec((D, D))],
    out_specs=_row_spec(D),
    out_shape=jax.ShapeDtypeStruct((N, D), jnp.float32),
)

_scale = pl.pallas_call(
    _scale_body,
    grid=(N // _RB,),
    in_specs=[_row_spec(D), _row_spec(2)],
    out_specs=[_row_spec(D), _row_spec(1)],
    out_shape=[
        jax.ShapeDtypeStruct((N, D), jnp.float32),
        jax.ShapeDtypeStruct((N, 1), jnp.float32),
    ],
)

_mid = pl.pallas_call(
    _mid_body,
    grid=(N // _RB,),
    in_specs=[
        _acc_spec(), _row_spec(D), _row_spec(1),
        _const_spec((1, D)), _const_spec((D, D)),
    ],
    out_specs=_row_spec(D),
    out_shape=jax.ShapeDtypeStruct((N, D), jnp.float32),
)

_fin = pl.pallas_call(
    _fin_body,
    grid=(N // _RB,),
    in_specs=[
        _acc_spec(), _row_spec(D), _row_spec(1), _const_spec((1, D)),
    ],
    out_specs=_row_spec(D),
    out_shape=jax.ShapeDtypeStruct((N, D), jnp.float32),
)


def kernel(x, edge_index, W1, b1, W2, b2):
    ei = edge_index.astype(jnp.int32)
    idx3 = ei.reshape(2, NCHT, CH).transpose(1, 0, 2)

    deg2 = _deg_kernel(ei)
    u1 = _mm0(x, W1)

    h1p, dinv = _scale(u1, deg2.T)
    acc1 = _edge_kernel(h1p, idx3)
    h2p = _mid(acc1, h1p, dinv, b1.reshape(1, D), W2)
    acc2 = _edge_kernel(h2p, idx3)
    return _fin(acc2, h2p, dinv, b2.reshape(1, D))
